# Initial kernel scaffold; baseline (speedup 1.0000x reference)
import jax
import jax.numpy as jnp
from jax.experimental import pallas as pl


def kernel(inputs, W_shared, b_shared, W_hm, b_hm, W_cls, b_cls, W_pos, b_pos, W_q, W_k, W_v, W_o, W_ff1, b_ff1, W_ff2, b_ff2, W_center, b_center, W_height, b_height, W_dim, b_dim, W_rot, b_rot, W_iou, b_iou, W_hm2, b_hm2):
    raise NotImplementedError("write your pallas kernel here")


# speedup vs baseline: 75.2878x; 75.2878x over previous
import numpy as np
import jax
import jax.numpy as jnp
from jax.experimental import pallas as pl
from jax.experimental.pallas import tpu as pltpu

X, Y = 192, 132
C = 128
NC = 6
NP = 500
R = 10
K1 = 2 * R + 1
K2 = K1 * K1
FF = 256
HW = X * Y
YP = Y + 2
TIL = 24
NG1 = X // TIL
OUTR = TIL * YP
PADL = 2048
LROWS = PADL + HW + PADL
T2 = 10
NG2 = NP // T2
NH = 15


def _conv_kernel(xp_ref, wt_ref, bs_ref, feat_ref):
    g = pl.program_id(0)
    x0 = g * TIL
    f32 = jnp.float32
    acc = jnp.zeros((OUTR, C), f32)
    for t in range(9):
        dx, dy = t // 3 - 1, t % 3 - 1
        start = (x0 + 3 + dx) * YP + (1 + dy)
        acc = acc + jnp.dot(
            xp_ref[pl.ds(start, OUTR), :].astype(jnp.bfloat16), wt_ref[t],
            preferred_element_type=f32)
    feat_ref[...] = jnp.maximum(acc + bs_ref[...], 0.0)


def _attn_kernel(idx_sref, cls_sref, lidar_ref, idxv_ref,
                 wpos_ref, bpos_ref, wclsT_ref, bcls_ref,
                 wq_ref, wk_ref, wv_ref, wo_ref,
                 wff1_ref, bff1_ref, wff2_ref, bff2_ref,
                 whead_ref, bhead_ref, out_ref, keys_ref, h_ref):
    t = pl.program_id(0)
    f32 = jnp.float32
    bf16 = jnp.bfloat16
    sqrt_c = np.float32(np.sqrt(C))

    def bf(x):
        return x.astype(bf16)

    def bmm(x, w):
        return jnp.dot(bf(x), bf(w), preferred_element_type=f32)

    def body(j, carry):
        idx = idx_sref[t * T2 + j]
        tx = idx // X
        ty = idx % Y
        base = (tx - R) * X + (ty - R)

        def gi(i, c):
            keys_ref[pl.ds(j * K2 + i * K1, K1), :] = (
                lidar_ref[pl.ds(PADL + base + i * X, K1), :])
            return c
        jax.lax.fori_loop(0, K1, gi, 0)

        m = jax.lax.broadcasted_iota(jnp.int32, (K2, 1), 0)
        flat = base + (m // K1) * X + (m % K1)
        invalid = (flat < 0) | (flat >= HW)
        cf = jnp.clip(flat, 0, HW - 1)
        wp0 = bf(wpos_ref[0:1, :]).astype(f32)
        wp1 = bf(wpos_ref[1:2, :]).astype(f32)
        px = bf((cf // Y).astype(f32) + 0.5).astype(f32)
        py = bf((cf % Y).astype(f32) + 0.5).astype(f32)
        posk = px * wp0 + py * wp1 + bpos_ref[...]

        cls = cls_sref[t * T2 + j]
        onehot = (jax.lax.broadcasted_iota(jnp.int32, (1, NC), 1)
                  == cls).astype(f32)
        qrow = lidar_ref[pl.ds(PADL + idx, 1), :]
        qt = qrow + bmm(onehot, wclsT_ref[...]) + bcls_ref[...]
        qxf = bf((idx // Y).astype(f32) + 0.5).astype(f32)
        qyf = bf((idx % Y).astype(f32) + 0.5).astype(f32)
        posq = qxf * wp0 + qyf * wp1 + bpos_ref[...]

        keys = keys_ref[pl.ds(j * K2, K2), :]
        q = bmm(qt + posq, wq_ref[...])
        k = bmm(keys + posk, wk_ref[...])
        v = bmm(keys, wv_ref[...])
        s = jnp.sum(bf(k).astype(f32) * bf(q).astype(f32),
                    axis=1, keepdims=True) / sqrt_c
        s = jnp.where(invalid, -1e9, s)
        smax = jnp.max(s, axis=0, keepdims=True)
        e = jnp.exp(s - smax)
        a = e / jnp.sum(e, axis=0, keepdims=True)
        av = jnp.sum(bf(v).astype(f32) * bf(a).astype(f32),
                     axis=0, keepdims=True)
        h = qt + bmm(av, wo_ref[...])
        h_ref[pl.ds(j, 1), :] = h
        return carry

    jax.lax.fori_loop(0, T2, body, 0)

    hs = h_ref[...]
    ff = bmm(jnp.maximum(bmm(hs, wff1_ref[...]) + bff1_ref[...], 0.0),
             wff2_ref[...])
    h2 = hs + (ff + bff2_ref[...])
    res = bmm(h2, whead_ref[...]) + bhead_ref[...]
    idxs = idxv_ref[0]
    qx = (idxs // Y).astype(f32) + 0.5
    qy = (idxs % Y).astype(f32) + 0.5
    lane = jax.lax.broadcasted_iota(jnp.int32, (T2, NH), 1)
    res = res + jnp.where(lane == 0, qx, 0.0) + jnp.where(lane == 1, qy, 0.0)
    out_ref[0] = res


@jax.jit
def kernel(inputs, W_shared, b_shared, W_hm, b_hm, W_cls, b_cls, W_pos, b_pos,
           W_q, W_k, W_v, W_o, W_ff1, b_ff1, W_ff2, b_ff2, W_center, b_center,
           W_height, b_height, W_dim, b_dim, W_rot, b_rot, W_iou, b_iou,
           W_hm2, b_hm2):
    f32 = jnp.float32
    x = inputs[0].transpose(1, 2, 0)
    xp = jnp.pad(x, ((3, 5), (1, 1), (0, 0))).reshape(-1, C)
    w_taps = W_shared.transpose(2, 3, 1, 0).reshape(9, C, C).astype(jnp.bfloat16)

    feat = pl.pallas_call(
        _conv_kernel,
        grid=(NG1,),
        in_specs=[
            pl.BlockSpec((200 * YP, C), lambda g: (0, 0)),
            pl.BlockSpec((9, C, C), lambda g: (0, 0, 0)),
            pl.BlockSpec((1, C), lambda g: (0, 0)),
        ],
        out_specs=pl.BlockSpec((OUTR, C), lambda g: (g, 0)),
        out_shape=jax.ShapeDtypeStruct((X * YP, C), f32),
    )(xp, w_taps, b_shared.reshape(1, C))

    def conv2d(v, w, b):
        o = jax.lax.conv_general_dilated(
            v, w, (1, 1), [(1, 1), (1, 1)],
            dimension_numbers=('NCHW', 'OIHW', 'NCHW'))
        return o + b[None, :, None, None]

    lidar_feat = jax.nn.relu(conv2d(inputs, W_shared, b_shared))
    heatmap = jax.nn.sigmoid(conv2d(lidar_feat, W_hm, b_hm))
    inner = jax.lax.reduce_window(heatmap, -jnp.inf, jax.lax.max,
                                  (1, 1, 3, 3), (1, 1, 1, 1), 'VALID')
    local_max = jnp.zeros_like(heatmap).at[:, :, 1:-1, 1:-1].set(inner)
    local_max = (local_max.at[:, 3].set(heatmap[:, 3])
                 .at[:, 4].set(heatmap[:, 4]).at[:, 5].set(heatmap[:, 5]))
    sup_hw = (heatmap * (heatmap == local_max)).reshape(-1)
    _, top = jax.lax.top_k(sup_hw[None], NP)
    top_cls = (top[0] // HW).astype(jnp.int32)
    top_idx = (top[0] % HW).astype(jnp.int32)

    lidar = feat.reshape(X, YP, C)[:, :Y].reshape(HW, C)
    lidar_p = jnp.pad(lidar, ((PADL, PADL), (0, 0)))

    w_heads = jnp.concatenate(
        [W_center, W_height, W_dim, W_rot, W_iou, W_hm2], axis=1)
    b_heads = jnp.concatenate(
        [b_center, b_height, b_dim, b_rot, b_iou, b_hm2]).reshape(1, NH)

    grid_spec = pltpu.PrefetchScalarGridSpec(
        num_scalar_prefetch=2,
        grid=(NG2,),
        in_specs=[
            pl.BlockSpec((LROWS, C), lambda g, *_: (0, 0)),
            pl.BlockSpec((1, T2, 1), lambda g, *_: (g, 0, 0)),
            pl.BlockSpec((2, C), lambda g, *_: (0, 0)),
            pl.BlockSpec((1, C), lambda g, *_: (0, 0)),
            pl.BlockSpec((NC, C), lambda g, *_: (0, 0)),
            pl.BlockSpec((1, C), lambda g, *_: (0, 0)),
            pl.BlockSpec((C, C), lambda g, *_: (0, 0)),
            pl.BlockSpec((C, C), lambda g, *_: (0, 0)),
            pl.BlockSpec((C, C), lambda g, *_: (0, 0)),
            pl.BlockSpec((C, C), lambda g, *_: (0, 0)),
            pl.BlockSpec((C, FF), lambda g, *_: (0, 0)),
            pl.BlockSpec((1, FF), lambda g, *_: (0, 0)),
            pl.BlockSpec((FF, C), lambda g, *_: (0, 0)),
            pl.BlockSpec((1, C), lambda g, *_: (0, 0)),
            pl.BlockSpec((C, NH), lambda g, *_: (0, 0)),
            pl.BlockSpec((1, NH), lambda g, *_: (0, 0)),
        ],
        out_specs=pl.BlockSpec((1, T2, NH), lambda g, *_: (g, 0, 0)),
        scratch_shapes=[
            pltpu.VMEM((T2 * K2, C), f32),
            pltpu.VMEM((T2, C), f32),
        ],
    )
    res = pl.pallas_call(
        _attn_kernel,
        grid_spec=grid_spec,
        out_shape=jax.ShapeDtypeStruct((NG2, T2, NH), f32),
    )(top_idx, top_cls, lidar_p, top_idx.reshape(NG2, T2, 1),
      W_pos, b_pos.reshape(1, C), W_cls.T, b_cls.reshape(1, C),
      W_q, W_k, W_v, W_o, W_ff1, b_ff1.reshape(1, FF), W_ff2,
      b_ff2.reshape(1, C), w_heads, b_heads)

    r = res.reshape(NP, NH).T[None]
    return (r[:, 0:2], r[:, 2:3], r[:, 3:6], r[:, 6:8], r[:, 8:9], r[:, 9:15])
